# baseline (speedup 1.0000x reference)
import jax
import jax.numpy as jnp
from jax import lax
from jax.experimental import pallas as pl
from jax.experimental.pallas import tpu as pltpu
from jax.experimental.pallas import tpu_sc as plsc

NC = 2
NS = 16
L = 16
NW = NC * NS

B = 16384
D = 32
NU = 1000000
NI = 1000000
BPW = B // NW
CH = 128
NCH = BPW // CH
NG = CH // L


def _bpr_body(user_h, pos_h, neg_h, ut_h, it_h, pos_out_h, neg_out_h,
              uidx, pidx, nidx, uixm, pixm, nixm,
              ubuf, pbuf, nbuf, outp, outn, sem):
    wid = lax.axis_index("s") * NC + lax.axis_index("c")
    base = wid * BPW

    du = pltpu.async_copy(user_h.at[pl.ds(base, BPW)], uidx, sem)
    dp = pltpu.async_copy(pos_h.at[pl.ds(base, BPW)], pidx, sem)
    dn = pltpu.async_copy(neg_h.at[pl.ds(base, BPW)], nidx, sem)
    du.wait()
    dp.wait()
    dn.wait()

    def chunk_body(c0, carry):
        cb = pl.multiple_of(c0 * CH, CH)

        def build(g, carry2):
            sl = pl.ds(pl.multiple_of(g * L, L), L)
            csl = pl.ds(pl.multiple_of(cb + g * L, L), L)
            ru = uidx[csl]
            rp = pidx[csl]
            rn = nidx[csl]
            for c in range(D):
                uixm[c, sl] = ru + c * NU
                pixm[c, sl] = rp + c * NI
                nixm[c, sl] = rn + c * NI
            return carry2

        lax.fori_loop(0, NG, build, 0)

        for c in range(D):
            pltpu.async_copy(ut_h.at[uixm.at[c]], ubuf.at[c], sem)
            pltpu.async_copy(it_h.at[pixm.at[c]], pbuf.at[c], sem)
            pltpu.async_copy(it_h.at[nixm.at[c]], nbuf.at[c], sem)
        for c in range(D):
            pltpu.make_async_copy(ut_h.at[uixm.at[0]], ubuf.at[0], sem).wait()
            pltpu.make_async_copy(it_h.at[pixm.at[0]], pbuf.at[0], sem).wait()
            pltpu.make_async_copy(it_h.at[nixm.at[0]], nbuf.at[0], sem).wait()

        def dots(g, carry2):
            sl = pl.ds(pl.multiple_of(g * L, L), L)
            accp = jnp.zeros((L,), jnp.float32)
            accn = jnp.zeros((L,), jnp.float32)
            for c in range(D):
                uv = ubuf[c, sl]
                accp = accp + uv * pbuf[c, sl]
                accn = accn + uv * nbuf[c, sl]
            osl = pl.ds(pl.multiple_of(cb + g * L, L), L)
            outp[osl] = accp
            outn[osl] = accn
            return carry2

        lax.fori_loop(0, NG, dots, 0)
        return carry

    lax.fori_loop(0, NCH, chunk_body, 0)

    pltpu.sync_copy(outp, pos_out_h.at[pl.ds(base, BPW)])
    pltpu.sync_copy(outn, neg_out_h.at[pl.ds(base, BPW)])


@jax.jit
def _bpr(user, pos_item, neg_item, user_table, item_table):
    ut1 = user_table.T.reshape(-1)
    it1 = item_table.T.reshape(-1)
    run = pl.kernel(
        _bpr_body,
        out_type=(jax.ShapeDtypeStruct((B,), jnp.float32),
                  jax.ShapeDtypeStruct((B,), jnp.float32)),
        mesh=plsc.VectorSubcoreMesh(core_axis_name="c", subcore_axis_name="s"),
        scratch_types=[
            pltpu.VMEM((BPW,), jnp.int32),
            pltpu.VMEM((BPW,), jnp.int32),
            pltpu.VMEM((BPW,), jnp.int32),
            pltpu.VMEM((D, CH), jnp.int32),
            pltpu.VMEM((D, CH), jnp.int32),
            pltpu.VMEM((D, CH), jnp.int32),
            pltpu.VMEM((D, CH), jnp.float32),
            pltpu.VMEM((D, CH), jnp.float32),
            pltpu.VMEM((D, CH), jnp.float32),
            pltpu.VMEM((BPW,), jnp.float32),
            pltpu.VMEM((BPW,), jnp.float32),
            pltpu.SemaphoreType.DMA,
        ],
        compiler_params=pltpu.CompilerParams(
            needs_layout_passes=False, use_tc_tiling_on_sc=False),
    )
    return run(user, pos_item, neg_item, ut1, it1)


def kernel(user, pos_item, neg_item, user_table, item_table):
    return _bpr(user, pos_item, neg_item, user_table, item_table)


# speedup vs baseline: 3.7320x; 3.7320x over previous
import jax
import jax.numpy as jnp
from jax import lax
from jax.experimental import pallas as pl
from jax.experimental.pallas import tpu as pltpu
from jax.experimental.pallas import tpu_sc as plsc

NC = 2
NS = 16
L = 16
NW = NC * NS

B = 16384
D = 32
NU = 1000000
NI = 1000000
NT = NU // 128 + 1
CB = D // 8
KT = 13
NK = (NT + KT - 1) // KT
ROWS = CB * NT * 8

BPW = B // NW
CH = 128
NCH = BPW // CH
NG = CH // L


def _detile_body(ut_ref, it_ref, uo_ref, io_ref):
    for src, dst in ((ut_ref, uo_ref), (it_ref, io_ref)):
        x = src[...]
        dst[...] = x.reshape(8, KT, 128).swapaxes(0, 1).reshape(KT * 8, 128)


def _bpr_gather_body(user_h, pos_h, neg_h, ut_h, it_h, pos_out_h, neg_out_h,
                     uidx, pidx, nidx, uixm, pixm, nixm,
                     ubuf, pbuf, nbuf, outp, outn, sem):
    wid = lax.axis_index("s") * NC + lax.axis_index("c")
    base = wid * BPW

    du = pltpu.async_copy(user_h.at[pl.ds(base, BPW)], uidx, sem)
    dp = pltpu.async_copy(pos_h.at[pl.ds(base, BPW)], pidx, sem)
    dn = pltpu.async_copy(neg_h.at[pl.ds(base, BPW)], nidx, sem)
    du.wait()
    dp.wait()
    dn.wait()

    def chunk_body(c0, carry):
        cb = pl.multiple_of(c0 * CH, CH)

        def build(g, carry2):
            sl = pl.ds(pl.multiple_of(g * L, L), L)
            csl = pl.ds(pl.multiple_of(cb + g * L, L), L)
            ru = uidx[csl]
            rp = pidx[csl]
            rn = nidx[csl]
            tu = (ru >> 7) * 1024 + (ru & 127)
            tp = (rp >> 7) * 1024 + (rp & 127)
            tn = (rn >> 7) * 1024 + (rn & 127)
            for c in range(D):
                cw = (c >> 3) * (NT * 1024) + (c & 7) * 128
                uixm[c, sl] = tu + cw
                pixm[c, sl] = tp + cw
                nixm[c, sl] = tn + cw
            return carry2

        lax.fori_loop(0, NG, build, 0)

        for c in range(D):
            pltpu.async_copy(ut_h.at[uixm.at[c]], ubuf.at[c], sem)
            pltpu.async_copy(it_h.at[pixm.at[c]], pbuf.at[c], sem)
            pltpu.async_copy(it_h.at[nixm.at[c]], nbuf.at[c], sem)
        for c in range(D):
            pltpu.make_async_copy(ut_h.at[uixm.at[0]], ubuf.at[0], sem).wait()
            pltpu.make_async_copy(it_h.at[pixm.at[0]], pbuf.at[0], sem).wait()
            pltpu.make_async_copy(it_h.at[nixm.at[0]], nbuf.at[0], sem).wait()

        def dots(g, carry2):
            sl = pl.ds(pl.multiple_of(g * L, L), L)
            accp = jnp.zeros((L,), jnp.float32)
            accn = jnp.zeros((L,), jnp.float32)
            for c in range(D):
                uv = ubuf[c, sl]
                accp = accp + uv * pbuf[c, sl]
                accn = accn + uv * nbuf[c, sl]
            osl = pl.ds(pl.multiple_of(cb + g * L, L), L)
            outp[osl] = accp
            outn[osl] = accn
            return carry2

        lax.fori_loop(0, NG, dots, 0)
        return carry

    lax.fori_loop(0, NCH, chunk_body, 0)

    pltpu.sync_copy(outp, pos_out_h.at[pl.ds(base, BPW)])
    pltpu.sync_copy(outn, neg_out_h.at[pl.ds(base, BPW)])


@jax.jit
def _bpr(user, pos_item, neg_item, user_table, item_table):
    ut = user_table.T
    it = item_table.T
    utd, itd = pl.pallas_call(
        _detile_body,
        grid=(CB, NK),
        in_specs=[
            pl.BlockSpec((8, KT * 128), lambda cb, k: (cb, k)),
            pl.BlockSpec((8, KT * 128), lambda cb, k: (cb, k)),
        ],
        out_specs=[
            pl.BlockSpec((KT * 8, 128), lambda cb, k: (cb * NK + k, 0)),
            pl.BlockSpec((KT * 8, 128), lambda cb, k: (cb * NK + k, 0)),
        ],
        out_shape=[
            jax.ShapeDtypeStruct((ROWS, 128), jnp.float32),
            jax.ShapeDtypeStruct((ROWS, 128), jnp.float32),
        ],
    )(ut, it)

    run = pl.kernel(
        _bpr_gather_body,
        out_type=(jax.ShapeDtypeStruct((B,), jnp.float32),
                  jax.ShapeDtypeStruct((B,), jnp.float32)),
        mesh=plsc.VectorSubcoreMesh(core_axis_name="c", subcore_axis_name="s"),
        scratch_types=[
            pltpu.VMEM((BPW,), jnp.int32),
            pltpu.VMEM((BPW,), jnp.int32),
            pltpu.VMEM((BPW,), jnp.int32),
            pltpu.VMEM((D, CH), jnp.int32),
            pltpu.VMEM((D, CH), jnp.int32),
            pltpu.VMEM((D, CH), jnp.int32),
            pltpu.VMEM((D, CH), jnp.float32),
            pltpu.VMEM((D, CH), jnp.float32),
            pltpu.VMEM((D, CH), jnp.float32),
            pltpu.VMEM((BPW,), jnp.float32),
            pltpu.VMEM((BPW,), jnp.float32),
            pltpu.SemaphoreType.DMA,
        ],
        compiler_params=pltpu.CompilerParams(
            needs_layout_passes=False, use_tc_tiling_on_sc=False),
    )
    return run(user, pos_item, neg_item, utd.reshape(-1), itd.reshape(-1))


def kernel(user, pos_item, neg_item, user_table, item_table):
    return _bpr(user, pos_item, neg_item, user_table, item_table)


# speedup vs baseline: 15.4241x; 4.1329x over previous
import jax
import jax.numpy as jnp
from jax import lax
from jax.experimental import pallas as pl
from jax.experimental.pallas import tpu as pltpu
from jax.experimental.pallas import tpu_sc as plsc

NC = 2
NS = 16
L = 16
NW = NC * NS

B = 16384
D = 32
NU = 1000000
NI = 1000000
NT = NU // 128 + 1
CB = D // 8
KT = 128
NK = (NT + KT - 1) // KT
PT = NK * KT
ROWS = CB * PT * 8

BPW = B // NW
CH = 128
NCH = BPW // CH
NG = CH // L


def _detile_body(ut_ref, it_ref, uo_ref, io_ref):
    for src, dst in ((ut_ref, uo_ref), (it_ref, io_ref)):
        for j in range(KT):
            dst[pl.ds(j * 8, 8), :] = src[:, pl.ds(j * 128, 128)]


def _bpr_gather_body(user_h, pos_h, neg_h, ut_h, it_h, pos_out_h, neg_out_h,
                     uidx, pidx, nidx, uixm, pixm, nixm,
                     ubuf, pbuf, nbuf, outp, outn, sem):
    wid = lax.axis_index("s") * NC + lax.axis_index("c")
    base = wid * BPW

    du = pltpu.async_copy(user_h.at[pl.ds(base, BPW)], uidx, sem)
    dp = pltpu.async_copy(pos_h.at[pl.ds(base, BPW)], pidx, sem)
    dn = pltpu.async_copy(neg_h.at[pl.ds(base, BPW)], nidx, sem)
    du.wait()
    dp.wait()
    dn.wait()

    def chunk_body(c0, carry):
        cb = pl.multiple_of(c0 * CH, CH)

        def build(g, carry2):
            sl = pl.ds(pl.multiple_of(g * L, L), L)
            csl = pl.ds(pl.multiple_of(cb + g * L, L), L)
            ru = uidx[csl]
            rp = pidx[csl]
            rn = nidx[csl]
            tu = (ru >> 7) * 1024 + (ru & 127)
            tp = (rp >> 7) * 1024 + (rp & 127)
            tn = (rn >> 7) * 1024 + (rn & 127)
            for c in range(D):
                cw = (c >> 3) * (PT * 1024) + (c & 7) * 128
                uixm[c, sl] = tu + cw
                pixm[c, sl] = tp + cw
                nixm[c, sl] = tn + cw
            return carry2

        lax.fori_loop(0, NG, build, 0)

        for c in range(D):
            pltpu.async_copy(ut_h.at[uixm.at[c]], ubuf.at[c], sem)
            pltpu.async_copy(it_h.at[pixm.at[c]], pbuf.at[c], sem)
            pltpu.async_copy(it_h.at[nixm.at[c]], nbuf.at[c], sem)
        for c in range(D):
            pltpu.make_async_copy(ut_h.at[uixm.at[0]], ubuf.at[0], sem).wait()
            pltpu.make_async_copy(it_h.at[pixm.at[0]], pbuf.at[0], sem).wait()
            pltpu.make_async_copy(it_h.at[nixm.at[0]], nbuf.at[0], sem).wait()

        def dots(g, carry2):
            sl = pl.ds(pl.multiple_of(g * L, L), L)
            accp = jnp.zeros((L,), jnp.float32)
            accn = jnp.zeros((L,), jnp.float32)
            for c in range(D):
                uv = ubuf[c, sl]
                accp = accp + uv * pbuf[c, sl]
                accn = accn + uv * nbuf[c, sl]
            osl = pl.ds(pl.multiple_of(cb + g * L, L), L)
            outp[osl] = accp
            outn[osl] = accn
            return carry2

        lax.fori_loop(0, NG, dots, 0)
        return carry

    lax.fori_loop(0, NCH, chunk_body, 0)

    pltpu.sync_copy(outp, pos_out_h.at[pl.ds(base, BPW)])
    pltpu.sync_copy(outn, neg_out_h.at[pl.ds(base, BPW)])


@jax.jit
def _bpr(user, pos_item, neg_item, user_table, item_table):
    ut = user_table.T
    it = item_table.T
    utd, itd = pl.pallas_call(
        _detile_body,
        grid=(CB, NK),
        in_specs=[
            pl.BlockSpec((8, KT * 128), lambda cb, k: (cb, k)),
            pl.BlockSpec((8, KT * 128), lambda cb, k: (cb, k)),
        ],
        out_specs=[
            pl.BlockSpec((KT * 8, 128), lambda cb, k: (cb * NK + k, 0)),
            pl.BlockSpec((KT * 8, 128), lambda cb, k: (cb * NK + k, 0)),
        ],
        out_shape=[
            jax.ShapeDtypeStruct((ROWS, 128), jnp.float32),
            jax.ShapeDtypeStruct((ROWS, 128), jnp.float32),
        ],
    )(ut, it)

    run = pl.kernel(
        _bpr_gather_body,
        out_type=(jax.ShapeDtypeStruct((B,), jnp.float32),
                  jax.ShapeDtypeStruct((B,), jnp.float32)),
        mesh=plsc.VectorSubcoreMesh(core_axis_name="c", subcore_axis_name="s"),
        scratch_types=[
            pltpu.VMEM((BPW,), jnp.int32),
            pltpu.VMEM((BPW,), jnp.int32),
            pltpu.VMEM((BPW,), jnp.int32),
            pltpu.VMEM((D, CH), jnp.int32),
            pltpu.VMEM((D, CH), jnp.int32),
            pltpu.VMEM((D, CH), jnp.int32),
            pltpu.VMEM((D, CH), jnp.float32),
            pltpu.VMEM((D, CH), jnp.float32),
            pltpu.VMEM((D, CH), jnp.float32),
            pltpu.VMEM((BPW,), jnp.float32),
            pltpu.VMEM((BPW,), jnp.float32),
            pltpu.SemaphoreType.DMA,
        ],
        compiler_params=pltpu.CompilerParams(
            needs_layout_passes=False, use_tc_tiling_on_sc=False),
    )
    return run(user, pos_item, neg_item, utd.reshape(-1), itd.reshape(-1))


def kernel(user, pos_item, neg_item, user_table, item_table):
    return _bpr(user, pos_item, neg_item, user_table, item_table)


# speedup vs baseline: 18.0720x; 1.1717x over previous
import jax
import jax.numpy as jnp
from jax import lax
from jax.experimental import pallas as pl
from jax.experimental.pallas import tpu as pltpu
from jax.experimental.pallas import tpu_sc as plsc

NC = 2
NS = 16
L = 16
NW = NC * NS

B = 16384
D = 32
BPW = B // NW
NBUF = 4
LANE = 128
NGRP = BPW // L


def _bpr_body(user_h, pos_h, neg_h, ut_h, it_h, pos_out_h, neg_out_h,
              uidx, pidx, nidx, ubuf, pbuf, nbuf, outp, outn, sem):
    wid = lax.axis_index("s") * NC + lax.axis_index("c")
    base = wid * BPW

    du = pltpu.async_copy(user_h.at[pl.ds(base, BPW)], uidx, sem)
    dp = pltpu.async_copy(pos_h.at[pl.ds(base, BPW)], pidx, sem)
    dn = pltpu.async_copy(neg_h.at[pl.ds(base, BPW)], nidx, sem)
    du.wait()
    dp.wait()
    dn.wait()

    c_lo = lax.iota(jnp.int32, L)
    c_hi = c_lo + L

    def fire(ru, rp, rn, slot):
        ou = pl.multiple_of((ru >> 7) * LANE, LANE)
        op = pl.multiple_of((rp >> 7) * LANE, LANE)
        on = pl.multiple_of((rn >> 7) * LANE, LANE)
        pltpu.async_copy(ut_h.at[:, pl.ds(ou, LANE)], ubuf.at[slot], sem)
        pltpu.async_copy(it_h.at[:, pl.ds(op, LANE)], pbuf.at[slot], sem)
        pltpu.async_copy(it_h.at[:, pl.ds(on, LANE)], nbuf.at[slot], sem)

    def drain():
        pltpu.make_async_copy(ut_h.at[:, pl.ds(0, LANE)], ubuf.at[0], sem).wait()
        pltpu.make_async_copy(it_h.at[:, pl.ds(0, LANE)], pbuf.at[0], sem).wait()
        pltpu.make_async_copy(it_h.at[:, pl.ds(0, LANE)], nbuf.at[0], sem).wait()

    vu0 = uidx[pl.ds(0, L)]
    vp0 = pidx[pl.ds(0, L)]
    vn0 = nidx[pl.ds(0, L)]
    for k in range(NBUF):
        fire(vu0[k], vp0[k], vn0[k], k)

    def group(g, carry):
        accp, accn = carry
        gb = pl.multiple_of(g * L, L)
        vu = uidx[pl.ds(gb, L)]
        vp = pidx[pl.ds(gb, L)]
        vn = nidx[pl.ds(gb, L)]
        nb = pl.multiple_of(jnp.minimum(gb + L, BPW - L), L)
        nvu = uidx[pl.ds(nb, L)]
        nvp = pidx[pl.ds(nb, L)]
        nvn = nidx[pl.ds(nb, L)]
        for k in range(L):
            i = gb + k
            drain()
            slot = jnp.full((L,), k % NBUF, jnp.int32)
            lu = jnp.full((L,), 0, jnp.int32) + (vu[k] & 127)
            lp = jnp.full((L,), 0, jnp.int32) + (vp[k] & 127)
            ln = jnp.full((L,), 0, jnp.int32) + (vn[k] & 127)
            u0 = plsc.load_gather(ubuf, [slot, c_lo, lu])
            u1 = plsc.load_gather(ubuf, [slot, c_hi, lu])
            p0 = plsc.load_gather(pbuf, [slot, c_lo, lp])
            p1 = plsc.load_gather(pbuf, [slot, c_hi, lp])
            n0 = plsc.load_gather(nbuf, [slot, c_lo, ln])
            n1 = plsc.load_gather(nbuf, [slot, c_hi, ln])
            sp = lax.reduce_sum_p.bind(u0 * p0 + u1 * p1, axes=(0,))
            sn = lax.reduce_sum_p.bind(u0 * n0 + u1 * n1, axes=(0,))
            accp = jnp.where(c_lo == k, sp, accp)
            accn = jnp.where(c_lo == k, sn, accn)
            if k == L - 1:
                outp[pl.ds(gb, L)] = accp
                outn[pl.ds(gb, L)] = accn
                accp = jnp.zeros((L,), jnp.float32)
                accn = jnp.zeros((L,), jnp.float32)
            if k + NBUF < L:
                ru, rp, rn = vu[k + NBUF], vp[k + NBUF], vn[k + NBUF]
            else:
                ru, rp, rn = nvu[k + NBUF - L], nvp[k + NBUF - L], nvn[k + NBUF - L]

            @pl.when(i + NBUF < BPW)
            def _():
                fire(ru, rp, rn, k % NBUF)

        return accp, accn

    zero = jnp.zeros((L,), jnp.float32)
    lax.fori_loop(0, NGRP, group, (zero, zero))

    pltpu.sync_copy(outp, pos_out_h.at[pl.ds(base, BPW)])
    pltpu.sync_copy(outn, neg_out_h.at[pl.ds(base, BPW)])


@jax.jit
def _bpr(user, pos_item, neg_item, user_table, item_table):
    run = pl.kernel(
        _bpr_body,
        out_type=(jax.ShapeDtypeStruct((B,), jnp.float32),
                  jax.ShapeDtypeStruct((B,), jnp.float32)),
        mesh=plsc.VectorSubcoreMesh(core_axis_name="c", subcore_axis_name="s"),
        scratch_types=[
            pltpu.VMEM((BPW,), jnp.int32),
            pltpu.VMEM((BPW,), jnp.int32),
            pltpu.VMEM((BPW,), jnp.int32),
            pltpu.VMEM((NBUF, D, LANE), jnp.float32),
            pltpu.VMEM((NBUF, D, LANE), jnp.float32),
            pltpu.VMEM((NBUF, D, LANE), jnp.float32),
            pltpu.VMEM((BPW,), jnp.float32),
            pltpu.VMEM((BPW,), jnp.float32),
            pltpu.SemaphoreType.DMA,
        ],
        compiler_params=pltpu.CompilerParams(needs_layout_passes=False),
    )
    return run(user, pos_item, neg_item, user_table.T, item_table.T)


def kernel(user, pos_item, neg_item, user_table, item_table):
    return _bpr(user, pos_item, neg_item, user_table, item_table)


# speedup vs baseline: 19.9297x; 1.1028x over previous
import jax
import jax.numpy as jnp
from jax import lax
from jax.experimental import pallas as pl
from jax.experimental.pallas import tpu as pltpu
from jax.experimental.pallas import tpu_sc as plsc

NC = 2
NS = 16
L = 16
NW = NC * NS

B = 16384
D = 32
NU = 1000000
NI = 1000000
NT = NU // 128 + 1
CB = D // 8
KT = 256
NK = (NT + KT - 1) // KT
PT = NK * KT
ROWS = CB * PT * 8

BPW = B // NW
CH = 128
NCH = BPW // CH
NG = CH // L


def _detile_body(ut_ref, it_ref, uo_ref, io_ref):
    for src, dst in ((ut_ref, uo_ref), (it_ref, io_ref)):
        for j in range(KT):
            dst[pl.ds(j * 8, 8), :] = src[:, pl.ds(j * 128, 128)]


def _bpr_gather_body(user_h, pos_h, neg_h, ut_h, it_h, pos_out_h, neg_out_h,
                     uidx, pidx, nidx, uixm, pixm, nixm,
                     ubuf, pbuf, nbuf, outp, outn, sem):
    wid = lax.axis_index("s") * NC + lax.axis_index("c")
    base = wid * BPW

    du = pltpu.async_copy(user_h.at[pl.ds(base, BPW)], uidx, sem)
    dp = pltpu.async_copy(pos_h.at[pl.ds(base, BPW)], pidx, sem)
    dn = pltpu.async_copy(neg_h.at[pl.ds(base, BPW)], nidx, sem)
    du.wait()
    dp.wait()
    dn.wait()

    def chunk_body(c0, carry):
        cb = pl.multiple_of(c0 * CH, CH)

        def build(g, carry2):
            sl = pl.ds(pl.multiple_of(g * L, L), L)
            csl = pl.ds(pl.multiple_of(cb + g * L, L), L)
            ru = uidx[csl]
            rp = pidx[csl]
            rn = nidx[csl]
            tu = (ru >> 7) * 1024 + (ru & 127)
            tp = (rp >> 7) * 1024 + (rp & 127)
            tn = (rn >> 7) * 1024 + (rn & 127)
            for c in range(D):
                cw = (c >> 3) * (PT * 1024) + (c & 7) * 128
                uixm[c, sl] = tu + cw
                pixm[c, sl] = tp + cw
                nixm[c, sl] = tn + cw
            return carry2

        lax.fori_loop(0, NG, build, 0)

        for c in range(D):
            pltpu.async_copy(ut_h.at[uixm.at[c]], ubuf.at[c], sem)
            pltpu.async_copy(it_h.at[pixm.at[c]], pbuf.at[c], sem)
            pltpu.async_copy(it_h.at[nixm.at[c]], nbuf.at[c], sem)
        for c in range(D):
            pltpu.make_async_copy(ut_h.at[uixm.at[0]], ubuf.at[0], sem).wait()
            pltpu.make_async_copy(it_h.at[pixm.at[0]], pbuf.at[0], sem).wait()
            pltpu.make_async_copy(it_h.at[nixm.at[0]], nbuf.at[0], sem).wait()

        def dots(g, carry2):
            sl = pl.ds(pl.multiple_of(g * L, L), L)
            accp = jnp.zeros((L,), jnp.float32)
            accn = jnp.zeros((L,), jnp.float32)
            for c in range(D):
                uv = ubuf[c, sl]
                accp = accp + uv * pbuf[c, sl]
                accn = accn + uv * nbuf[c, sl]
            osl = pl.ds(pl.multiple_of(cb + g * L, L), L)
            outp[osl] = accp
            outn[osl] = accn
            return carry2

        lax.fori_loop(0, NG, dots, 0)
        return carry

    lax.fori_loop(0, NCH, chunk_body, 0)

    pltpu.sync_copy(outp, pos_out_h.at[pl.ds(base, BPW)])
    pltpu.sync_copy(outn, neg_out_h.at[pl.ds(base, BPW)])


@jax.jit
def _bpr(user, pos_item, neg_item, user_table, item_table):
    ut = user_table.T
    it = item_table.T
    utd, itd = pl.pallas_call(
        _detile_body,
        grid=(CB, NK),
        in_specs=[
            pl.BlockSpec((8, KT * 128), lambda cb, k: (cb, k)),
            pl.BlockSpec((8, KT * 128), lambda cb, k: (cb, k)),
        ],
        out_specs=[
            pl.BlockSpec((KT * 8, 128), lambda cb, k: (cb * NK + k, 0)),
            pl.BlockSpec((KT * 8, 128), lambda cb, k: (cb * NK + k, 0)),
        ],
        out_shape=[
            jax.ShapeDtypeStruct((ROWS, 128), jnp.float32),
            jax.ShapeDtypeStruct((ROWS, 128), jnp.float32),
        ],
    )(ut, it)

    run = pl.kernel(
        _bpr_gather_body,
        out_type=(jax.ShapeDtypeStruct((B,), jnp.float32),
                  jax.ShapeDtypeStruct((B,), jnp.float32)),
        mesh=plsc.VectorSubcoreMesh(core_axis_name="c", subcore_axis_name="s"),
        scratch_types=[
            pltpu.VMEM((BPW,), jnp.int32),
            pltpu.VMEM((BPW,), jnp.int32),
            pltpu.VMEM((BPW,), jnp.int32),
            pltpu.VMEM((D, CH), jnp.int32),
            pltpu.VMEM((D, CH), jnp.int32),
            pltpu.VMEM((D, CH), jnp.int32),
            pltpu.VMEM((D, CH), jnp.float32),
            pltpu.VMEM((D, CH), jnp.float32),
            pltpu.VMEM((D, CH), jnp.float32),
            pltpu.VMEM((BPW,), jnp.float32),
            pltpu.VMEM((BPW,), jnp.float32),
            pltpu.SemaphoreType.DMA,
        ],
        compiler_params=pltpu.CompilerParams(
            needs_layout_passes=False, use_tc_tiling_on_sc=False),
    )
    return run(user, pos_item, neg_item, utd.reshape(-1), itd.reshape(-1))


def kernel(user, pos_item, neg_item, user_table, item_table):
    return _bpr(user, pos_item, neg_item, user_table, item_table)
